# Initial kernel scaffold; baseline (speedup 1.0000x reference)
import jax
import jax.numpy as jnp
from jax.experimental import pallas as pl


def kernel(x, edge_index, edge_weight, params):
    raise NotImplementedError("write your pallas kernel here")


# speedup vs baseline: 4.2084x; 4.2084x over previous
import dataclasses
import functools

import jax
import jax.numpy as jnp
from jax import lax
from jax.experimental import pallas as pl
from jax.experimental.pallas import tpu as pltpu
from jax.experimental.pallas import tpu_sc as plsc

N = 10000
E = 320000
K = 5
F_IN = 128
H1 = 32
H2D = 16
PERIODS = 12

N_PAD = 10240
NW = 32
EPT = 10240
E_PAD = NW * EPT
BPT = EPT // 128
CHUNK = 512
NCHUNK = EPT // CHUNK
BN = 1024
GRID_N = N_PAD // BN

_mesh = plsc.VectorSubcoreMesh(core_axis_name="c", subcore_axis_name="s")

_sc_params = pltpu.CompilerParams(
    needs_layout_passes=False, use_tc_tiling_on_sc=False)


def _iota16():
    return lax.iota(jnp.int32, 16)


def _fullc(c):
    return jnp.full((16,), c, jnp.int32)


EB = E_PAD // 128


def _deg_call(row2, col2, ew2):
    @functools.partial(
        pl.kernel,
        out_type=jax.ShapeDtypeStruct((2 * N_PAD, 16), jnp.float32),
        mesh=_mesh,
        compiler_params=_sc_params,
        scratch_types=[
            pltpu.VMEM_SHARED((N_PAD, 16), jnp.float32),
            pltpu.VMEM((BPT, 128), jnp.int32),
            pltpu.VMEM((BPT, 128), jnp.int32),
            pltpu.VMEM((BPT, 128), jnp.float32),
            pltpu.VMEM((128, 16), jnp.float32),
        ],
    )
    def deg_kernel(row_hbm, col_hbm, ew_hbm, dpart_hbm,
                   acc, rowv, colv, ewv, sbuf):
        cid = lax.axis_index("c")
        sid = lax.axis_index("s")
        wid = cid * 16 + sid
        it = _iota16()
        zero = jnp.zeros((16,), jnp.float32)

        @pl.loop(0, 8)
        def _(g):
            rows = g * 16 + it
            for c in range(16):
                plsc.store_scatter(sbuf, [rows, _fullc(c)], zero)

        for k in range(5):
            pltpu.sync_copy(sbuf, acc.at[pl.ds(sid * 640 + k * 128, 128)])
        plsc.subcore_barrier()

        jbase = wid * BPT
        pltpu.sync_copy(row_hbm.at[pl.ds(jbase, BPT)], rowv)
        pltpu.sync_copy(col_hbm.at[pl.ds(jbase, BPT)], colv)
        pltpu.sync_copy(ew_hbm.at[pl.ds(jbase, BPT)], ewv)

        @pl.loop(0, BPT)
        def _(j):
            @pl.loop(0, 8)
            def _(g):
                s = pl.ds(g * 16, 16)
                rv = rowv.at[j][s]
                cv = colv.at[j][s]
                wv = ewv.at[j][s]
                wz = jnp.where(rv == cv, 0.0, wv)
                rows = g * 16 + it
                for c in range(16):
                    plsc.store_scatter(sbuf, [rows, _fullc(c)], wz)
            pltpu.sync_copy(sbuf, acc.at[rowv.at[j]], add=True)

        plsc.subcore_barrier()
        pltpu.sync_copy(acc.at[pl.ds(sid * 640, 640)],
                        dpart_hbm.at[pl.ds(cid * N_PAD + sid * 640, 640)])

    return deg_kernel(row2, col2, ew2)


def _norm_call(row2, col2, ew2, dis16):
    @functools.partial(
        pl.kernel,
        out_type=jax.ShapeDtypeStruct((EB, 128), jnp.float32),
        mesh=_mesh,
        compiler_params=_sc_params,
        scratch_types=[
            pltpu.VMEM((BPT, 128), jnp.int32),
            pltpu.VMEM((BPT, 128), jnp.int32),
            pltpu.VMEM((BPT, 128), jnp.float32),
            pltpu.VMEM((BPT, 128), jnp.float32),
            pltpu.VMEM((128, 16), jnp.float32),
            pltpu.VMEM((128, 16), jnp.float32),
        ],
    )
    def norm_kernel(row_hbm, col_hbm, ew_hbm, dis_hbm, norm_hbm,
                    rowv, colv, ewv, nout, drb, dcb):
        cid = lax.axis_index("c")
        sid = lax.axis_index("s")
        wid = cid * 16 + sid
        it = _iota16()
        z16 = jnp.zeros((16,), jnp.int32)

        jbase = wid * BPT
        pltpu.sync_copy(row_hbm.at[pl.ds(jbase, BPT)], rowv)
        pltpu.sync_copy(col_hbm.at[pl.ds(jbase, BPT)], colv)
        pltpu.sync_copy(ew_hbm.at[pl.ds(jbase, BPT)], ewv)

        @pl.loop(0, BPT)
        def _(j):
            pltpu.sync_copy(dis_hbm.at[rowv.at[j]], drb)
            pltpu.sync_copy(dis_hbm.at[colv.at[j]], dcb)

            @pl.loop(0, 8)
            def _(g):
                s = pl.ds(g * 16, 16)
                rv = rowv.at[j][s]
                cv = colv.at[j][s]
                wv = ewv.at[j][s]
                wz = jnp.where(rv == cv, 0.0, wv)
                rows = g * 16 + it
                dr = plsc.load_gather(drb, [rows, z16])
                dc = plsc.load_gather(dcb, [rows, z16])
                nout.at[j][s] = -(dr * wz * dc)

        pltpu.sync_copy(nout, norm_hbm.at[pl.ds(jbase, BPT)])

    return norm_kernel(row2, col2, ew2, dis16)


def _make_prop(w):
    @functools.partial(
        pl.kernel,
        out_type=jax.ShapeDtypeStruct((2 * N_PAD, w), jnp.float32),
        mesh=_mesh,
        compiler_params=_sc_params,
        scratch_types=[
            pltpu.VMEM_SHARED((N_PAD, w), jnp.float32),
            pltpu.VMEM((BPT, 128), jnp.int32),
            pltpu.VMEM((BPT, 128), jnp.int32),
            pltpu.VMEM((BPT, 128), jnp.float32),
            pltpu.VMEM((128, w), jnp.float32),
        ],
    )
    def prop_kernel(b_hbm, row_hbm, col_hbm, norm_hbm, pacc_hbm,
                    acc, rowv, colv, normv, gbuf):
        cid = lax.axis_index("c")
        sid = lax.axis_index("s")
        wid = cid * 16 + sid
        it = _iota16()
        zero = jnp.zeros((16,), jnp.float32)

        @pl.loop(0, 8)
        def _(g):
            rows = g * 16 + it
            for c in range(w):
                plsc.store_scatter(gbuf, [rows, _fullc(c)], zero)

        for k in range(5):
            pltpu.sync_copy(gbuf, acc.at[pl.ds(sid * 640 + k * 128, 128)])
        plsc.subcore_barrier()

        jbase = wid * BPT
        pltpu.sync_copy(row_hbm.at[pl.ds(jbase, BPT)], rowv)
        pltpu.sync_copy(col_hbm.at[pl.ds(jbase, BPT)], colv)
        pltpu.sync_copy(norm_hbm.at[pl.ds(jbase, BPT)], normv)

        @pl.loop(0, BPT)
        def _(j):
            pltpu.sync_copy(b_hbm.at[rowv.at[j]], gbuf)

            @pl.loop(0, 8)
            def _(g):
                nv = normv.at[j][pl.ds(g * 16, 16)]
                rows = g * 16 + it
                for c in range(w):
                    v = plsc.load_gather(gbuf, [rows, _fullc(c)])
                    plsc.store_scatter(gbuf, [rows, _fullc(c)], v * nv)

            pltpu.sync_copy(gbuf, acc.at[colv.at[j]], add=True)

        plsc.subcore_barrier()
        pltpu.sync_copy(acc.at[pl.ds(sid * 640, 640)],
                        pacc_hbm.at[pl.ds(cid * N_PAD + sid * 640, 640)])

    return prop_kernel


_prop64 = _make_prop(64)
_prop32 = _make_prop(32)


def _prep_call(x_pad, wcat, p0p1):
    def body(x_ref, w_ref, p0_ref, p1_ref, a_ref, dis_ref, diag_ref):
        for k in range(K):
            a_ref[k] = jnp.dot(x_ref[...], w_ref[k],
                               preferred_element_type=jnp.float32)
        d = p0_ref[...] + p1_ref[...]
        pos = d > 0.0
        dis_ref[...] = jnp.where(pos, lax.rsqrt(jnp.where(pos, d, 1.0)), 0.0)
        diag_ref[...] = jnp.where(pos, 0.0, -1.0)

    fin, wdim = wcat.shape[1], wcat.shape[2]
    return pl.pallas_call(
        body,
        grid=(GRID_N,),
        in_specs=[
            pl.BlockSpec((BN, fin), lambda i: (i, 0)),
            pl.BlockSpec((K, fin, wdim), lambda i: (0, 0, 0)),
            pl.BlockSpec((BN, 16), lambda i: (i, 0)),
            pl.BlockSpec((BN, 16), lambda i: (i + GRID_N, 0)),
        ],
        out_specs=[
            pl.BlockSpec((K, BN, wdim), lambda i: (0, i, 0)),
            pl.BlockSpec((BN, 16), lambda i: (i, 0)),
            pl.BlockSpec((BN, 16), lambda i: (i, 0)),
        ],
        out_shape=[
            jax.ShapeDtypeStruct((K, N_PAD, wdim), jnp.float32),
            jax.ShapeDtypeStruct((N_PAD, 16), jnp.float32),
            jax.ShapeDtypeStruct((N_PAD, 16), jnp.float32),
        ],
    )(x_pad, wcat, p0p1, p0p1)


def _axpy_call(a_all, kslot, w, pacc, diag16, b, bprev, alpha, beta):
    def body(a_ref, p0_ref, p1_ref, dg_ref, b_ref, bp_ref, o_ref):
        t = p0_ref[...] + p1_ref[...] + dg_ref[...][:, :1] * b_ref[...]
        o = a_ref[0] + alpha * t
        if beta:
            o = o + beta * bp_ref[...]
        o_ref[...] = o

    return pl.pallas_call(
        body,
        grid=(GRID_N,),
        in_specs=[
            pl.BlockSpec((1, BN, w), lambda i: (kslot, i, 0)),
            pl.BlockSpec((BN, w), lambda i: (i, 0)),
            pl.BlockSpec((BN, w), lambda i: (i + GRID_N, 0)),
            pl.BlockSpec((BN, 16), lambda i: (i, 0)),
            pl.BlockSpec((BN, w), lambda i: (i, 0)),
            pl.BlockSpec((BN, w), lambda i: (i, 0)),
        ],
        out_specs=pl.BlockSpec((BN, w), lambda i: (i, 0)),
        out_shape=jax.ShapeDtypeStruct((N_PAD, w), jnp.float32),
    )(a_all, pacc, pacc, diag16, b, bprev)


def _gate_call(a_all, w, h, pacc, diag16, b1, b2, bz, bh, wnext, bnext):
    def body(a_ref, p0_ref, p1_ref, dg_ref, b1_ref, b2_ref, bz_ref, bh_ref,
             wn_ref, bn_ref, o_ref):
        t = p0_ref[...] + p1_ref[...] + dg_ref[...][:, :1] * b1_ref[...]
        s = a_ref[0] + t - b2_ref[...]
        z = jax.nn.sigmoid(s[:, :h] + bz_ref[...])
        ht = jnp.tanh(s[:, h:] + bh_ref[...])
        hid = jnp.maximum((1.0 - z) * ht, 0.0)
        for k in range(wn_ref.shape[0]):
            o_ref[k] = jnp.dot(hid, wn_ref[k],
                               preferred_element_type=jnp.float32) + bn_ref[...]

    kn, ndim = wnext.shape[0], wnext.shape[2]
    return pl.pallas_call(
        body,
        grid=(GRID_N,),
        in_specs=[
            pl.BlockSpec((1, BN, w), lambda i: (0, i, 0)),
            pl.BlockSpec((BN, w), lambda i: (i, 0)),
            pl.BlockSpec((BN, w), lambda i: (i + GRID_N, 0)),
            pl.BlockSpec((BN, 16), lambda i: (i, 0)),
            pl.BlockSpec((BN, w), lambda i: (i, 0)),
            pl.BlockSpec((BN, w), lambda i: (i, 0)),
            pl.BlockSpec((1, h), lambda i: (0, 0)),
            pl.BlockSpec((1, h), lambda i: (0, 0)),
            pl.BlockSpec((kn, h, ndim), lambda i: (0, 0, 0)),
            pl.BlockSpec((1, ndim), lambda i: (0, 0)),
        ],
        out_specs=pl.BlockSpec((kn, BN, ndim), lambda i: (0, i, 0)),
        out_shape=jax.ShapeDtypeStruct((kn, N_PAD, ndim), jnp.float32),
    )(a_all, pacc, pacc, diag16, b1, b2, bz, bh, wnext, bnext)


def _cheb_layer(a_all, w, row2, col2, norm2, diag16):
    prop = _prop64 if w == 64 else _prop32
    b4 = a_all[4]
    p = prop(b4, row2, col2, norm2)
    b3 = _axpy_call(a_all, 3, w, p, diag16, b4, b4, 2.0, 0.0)
    p = prop(b3, row2, col2, norm2)
    b2 = _axpy_call(a_all, 2, w, p, diag16, b3, b4, 2.0, -1.0)
    p = prop(b2, row2, col2, norm2)
    b1 = _axpy_call(a_all, 1, w, p, diag16, b2, b3, 2.0, -1.0)
    p = prop(b1, row2, col2, norm2)
    return p, b1, b2


def kernel(x, edge_index, edge_weight, params):
    f32 = jnp.float32
    row = edge_index[0]
    col = edge_index[1]

    epad = E_PAD - E
    row2 = jnp.pad(row, (0, epad)).reshape(EB, 128)
    col2 = jnp.pad(col, (0, epad)).reshape(EB, 128)
    ew2 = jnp.pad(edge_weight, (0, epad)).reshape(EB, 128)
    x_pad = jnp.pad(x, ((0, N_PAD - N), (0, 0)))

    wcat1 = jnp.stack(
        [jnp.concatenate([params["xz1"]["W"][k], params["xh1"]["W"][k]], axis=1)
         for k in range(K)], axis=0)
    wcat2 = jnp.stack(
        [jnp.concatenate([params["xz2"]["W"][k], params["xh2"]["W"][k]], axis=1)
         for k in range(K)], axis=0)
    bz1 = (params["xz1"]["b"] + params["hz1"]["b"]).reshape(1, H1)
    bh1 = (params["xh1"]["b"] + params["hh1"]["b"]).reshape(1, H1)
    bz2 = (params["xz2"]["b"] + params["hz2"]["b"]).reshape(1, H2D)
    bh2 = (params["xh2"]["b"] + params["hh2"]["b"]).reshape(1, H2D)
    lin_w = params["lin_W"]
    lin_b = params["lin_b"].reshape(1, PERIODS)

    dpart = _deg_call(row2, col2, ew2)
    a1, dis16, diag16 = _prep_call(x_pad, wcat1, dpart)
    norm2 = _norm_call(row2, col2, ew2, dis16)

    p, b1, b2 = _cheb_layer(a1, 64, row2, col2, norm2, diag16)
    a2 = _gate_call(a1, 64, H1, p, diag16, b1, b2, bz1, bh1, wcat2,
                    jnp.zeros((1, wcat2.shape[2]), f32))

    p, b1, b2 = _cheb_layer(a2, 32, row2, col2, norm2, diag16)
    out_pad = _gate_call(a2, 32, H2D, p, diag16, b1, b2, bz2, bh2,
                         lin_w.reshape(1, H2D, PERIODS), lin_b)

    return out_pad[0, :N]


# speedup vs baseline: 5.4761x; 1.3012x over previous
import dataclasses
import functools

import jax
import jax.numpy as jnp
from jax import lax
from jax.experimental import pallas as pl
from jax.experimental.pallas import tpu as pltpu
from jax.experimental.pallas import tpu_sc as plsc

N = 10000
E = 320000
K = 5
F_IN = 128
H1 = 32
H2D = 16
PERIODS = 12

N_PAD = 10240
NW = 32
EPT = 10240
E_PAD = NW * EPT
BPT = EPT // 128
CHUNK = 512
NCHUNK = EPT // CHUNK
BN = 1024
GRID_N = N_PAD // BN

_mesh = plsc.VectorSubcoreMesh(core_axis_name="c", subcore_axis_name="s")

_sc_params = pltpu.CompilerParams(
    needs_layout_passes=False, use_tc_tiling_on_sc=False)


def _iota16():
    return lax.iota(jnp.int32, 16)


def _fullc(c):
    return jnp.full((16,), c, jnp.int32)


EB = E_PAD // 128


def _deg_call(row2, col2, ew2):
    @functools.partial(
        pl.kernel,
        out_type=jax.ShapeDtypeStruct((2 * N_PAD, 16), jnp.float32),
        mesh=_mesh,
        compiler_params=_sc_params,
        scratch_types=[
            pltpu.VMEM_SHARED((N_PAD, 16), jnp.float32),
            pltpu.VMEM((BPT, 128), jnp.int32),
            pltpu.VMEM((BPT, 128), jnp.int32),
            pltpu.VMEM((BPT, 128), jnp.float32),
            pltpu.VMEM((128, 16), jnp.float32),
        ],
    )
    def deg_kernel(row_hbm, col_hbm, ew_hbm, dpart_hbm,
                   acc, rowv, colv, ewv, sbuf):
        cid = lax.axis_index("c")
        sid = lax.axis_index("s")
        wid = cid * 16 + sid
        it = _iota16()
        zero = jnp.zeros((16,), jnp.float32)

        @pl.loop(0, 8)
        def _(g):
            rows = g * 16 + it
            for c in range(16):
                plsc.store_scatter(sbuf, [rows, _fullc(c)], zero)

        for k in range(5):
            pltpu.sync_copy(sbuf, acc.at[pl.ds(sid * 640 + k * 128, 128)])
        plsc.subcore_barrier()

        jbase = wid * BPT
        pltpu.sync_copy(row_hbm.at[pl.ds(jbase, BPT)], rowv)
        pltpu.sync_copy(col_hbm.at[pl.ds(jbase, BPT)], colv)
        pltpu.sync_copy(ew_hbm.at[pl.ds(jbase, BPT)], ewv)

        @pl.loop(0, BPT)
        def _(j):
            @pl.loop(0, 8)
            def _(g):
                s = pl.ds(g * 16, 16)
                rv = rowv.at[j][s]
                cv = colv.at[j][s]
                wv = ewv.at[j][s]
                wz = jnp.where(rv == cv, 0.0, wv)
                rows = g * 16 + it
                for c in range(16):
                    plsc.store_scatter(sbuf, [rows, _fullc(c)], wz)
            pltpu.sync_copy(sbuf, acc.at[rowv.at[j]], add=True)

        plsc.subcore_barrier()
        pltpu.sync_copy(acc.at[pl.ds(sid * 640, 640)],
                        dpart_hbm.at[pl.ds(cid * N_PAD + sid * 640, 640)])

    return deg_kernel(row2, col2, ew2)


def _norm_call(row2, col2, ew2, dis16):
    @functools.partial(
        pl.kernel,
        out_type=jax.ShapeDtypeStruct((EB, 128), jnp.float32),
        mesh=_mesh,
        compiler_params=_sc_params,
        scratch_types=[
            pltpu.VMEM((BPT, 128), jnp.int32),
            pltpu.VMEM((BPT, 128), jnp.int32),
            pltpu.VMEM((BPT, 128), jnp.float32),
            pltpu.VMEM((BPT, 128), jnp.float32),
            pltpu.VMEM((128, 16), jnp.float32),
            pltpu.VMEM((128, 16), jnp.float32),
        ],
    )
    def norm_kernel(row_hbm, col_hbm, ew_hbm, dis_hbm, norm_hbm,
                    rowv, colv, ewv, nout, drb, dcb):
        cid = lax.axis_index("c")
        sid = lax.axis_index("s")
        wid = cid * 16 + sid
        it = _iota16()
        z16 = jnp.zeros((16,), jnp.int32)

        jbase = wid * BPT
        pltpu.sync_copy(row_hbm.at[pl.ds(jbase, BPT)], rowv)
        pltpu.sync_copy(col_hbm.at[pl.ds(jbase, BPT)], colv)
        pltpu.sync_copy(ew_hbm.at[pl.ds(jbase, BPT)], ewv)

        @pl.loop(0, BPT)
        def _(j):
            pltpu.sync_copy(dis_hbm.at[rowv.at[j]], drb)
            pltpu.sync_copy(dis_hbm.at[colv.at[j]], dcb)

            @pl.loop(0, 8)
            def _(g):
                s = pl.ds(g * 16, 16)
                rv = rowv.at[j][s]
                cv = colv.at[j][s]
                wv = ewv.at[j][s]
                wz = jnp.where(rv == cv, 0.0, wv)
                rows = g * 16 + it
                dr = plsc.load_gather(drb, [rows, z16])
                dc = plsc.load_gather(dcb, [rows, z16])
                nout.at[j][s] = -(dr * wz * dc)

        pltpu.sync_copy(nout, norm_hbm.at[pl.ds(jbase, BPT)])

    return norm_kernel(row2, col2, ew2, dis16)


def _make_prop(w):
    nbuf = 2 if w == 64 else 4

    @functools.partial(
        pl.kernel,
        out_type=jax.ShapeDtypeStruct((2 * N_PAD, w), jnp.float32),
        mesh=_mesh,
        compiler_params=_sc_params,
        scratch_types=(
            [pltpu.VMEM_SHARED((N_PAD, w), jnp.float32),
             pltpu.VMEM((BPT, 128), jnp.int32),
             pltpu.VMEM((BPT, 128), jnp.int32),
             pltpu.VMEM((BPT, 128), jnp.float32)]
            + [pltpu.VMEM((128, w), jnp.float32)] * (2 * nbuf)
            + [pltpu.SemaphoreType.DMA] * (2 * nbuf)
        ),
    )
    def prop_kernel(b_hbm, row_hbm, col_hbm, norm_hbm, pacc_hbm,
                    acc, rowv, colv, normv, *bufsem):
        gbuf = bufsem[:nbuf]
        sbuf = bufsem[nbuf:2 * nbuf]
        gsem = bufsem[2 * nbuf:3 * nbuf]
        ssem = bufsem[3 * nbuf:]
        cid = lax.axis_index("c")
        sid = lax.axis_index("s")
        wid = cid * 16 + sid
        it = _iota16()
        zero = jnp.zeros((16,), jnp.float32)

        @pl.loop(0, 8)
        def _(g):
            rows = g * 16 + it
            for c in range(w):
                plsc.store_scatter(gbuf[0], [rows, _fullc(c)], zero)

        for k in range(5):
            pltpu.sync_copy(gbuf[0], acc.at[pl.ds(sid * 640 + k * 128, 128)])
        plsc.subcore_barrier()

        jbase = wid * BPT
        pltpu.sync_copy(row_hbm.at[pl.ds(jbase, BPT)], rowv)
        pltpu.sync_copy(col_hbm.at[pl.ds(jbase, BPT)], colv)
        pltpu.sync_copy(norm_hbm.at[pl.ds(jbase, BPT)], normv)

        def scale(j, src, dst):
            @pl.loop(0, 8)
            def _(g):
                nv = normv.at[j][pl.ds(g * 16, 16)]
                rows = g * 16 + it
                for c in range(w):
                    v = plsc.load_gather(src, [rows, _fullc(c)])
                    plsc.store_scatter(dst, [rows, _fullc(c)], v * nv)

        for b in range(nbuf):
            pltpu.async_copy(b_hbm.at[rowv.at[b]], gbuf[b], gsem[b])

        nit = BPT // nbuf

        @pl.loop(0, nit)
        def _(jj):
            for b in range(nbuf):
                j = jj * nbuf + b
                pltpu.make_async_copy(
                    b_hbm.at[rowv.at[j]], gbuf[b], gsem[b]).wait()

                @pl.when(jj > 0)
                def _():
                    pltpu.make_async_copy(
                        sbuf[b], acc.at[colv.at[j - nbuf]], ssem[b]).wait()

                scale(j, gbuf[b], sbuf[b])

                @pl.when(jj < nit - 1)
                def _():
                    pltpu.async_copy(
                        b_hbm.at[rowv.at[j + nbuf]], gbuf[b], gsem[b])

                pltpu.async_copy(sbuf[b], acc.at[colv.at[j]], ssem[b],
                                 add=True)

        for b in range(nbuf):
            pltpu.make_async_copy(
                sbuf[b], acc.at[colv.at[BPT - nbuf + b]], ssem[b]).wait()

        plsc.subcore_barrier()
        pltpu.sync_copy(acc.at[pl.ds(sid * 640, 640)],
                        pacc_hbm.at[pl.ds(cid * N_PAD + sid * 640, 640)])

    return prop_kernel


_prop64 = _make_prop(64)
_prop32 = _make_prop(32)


def _prep_call(x_pad, wcat, p0p1):
    def body(x_ref, w_ref, p0_ref, p1_ref, a_ref, dis_ref, diag_ref):
        for k in range(K):
            a_ref[k] = jnp.dot(x_ref[...], w_ref[k],
                               preferred_element_type=jnp.float32)
        d = p0_ref[...] + p1_ref[...]
        pos = d > 0.0
        dis_ref[...] = jnp.where(pos, lax.rsqrt(jnp.where(pos, d, 1.0)), 0.0)
        diag_ref[...] = jnp.where(pos, 0.0, -1.0)

    fin, wdim = wcat.shape[1], wcat.shape[2]
    return pl.pallas_call(
        body,
        grid=(GRID_N,),
        in_specs=[
            pl.BlockSpec((BN, fin), lambda i: (i, 0)),
            pl.BlockSpec((K, fin, wdim), lambda i: (0, 0, 0)),
            pl.BlockSpec((BN, 16), lambda i: (i, 0)),
            pl.BlockSpec((BN, 16), lambda i: (i + GRID_N, 0)),
        ],
        out_specs=[
            pl.BlockSpec((K, BN, wdim), lambda i: (0, i, 0)),
            pl.BlockSpec((BN, 16), lambda i: (i, 0)),
            pl.BlockSpec((BN, 16), lambda i: (i, 0)),
        ],
        out_shape=[
            jax.ShapeDtypeStruct((K, N_PAD, wdim), jnp.float32),
            jax.ShapeDtypeStruct((N_PAD, 16), jnp.float32),
            jax.ShapeDtypeStruct((N_PAD, 16), jnp.float32),
        ],
    )(x_pad, wcat, p0p1, p0p1)


def _axpy_call(a_all, kslot, w, pacc, diag16, b, bprev, alpha, beta):
    def body(a_ref, p0_ref, p1_ref, dg_ref, b_ref, bp_ref, o_ref):
        t = p0_ref[...] + p1_ref[...] + dg_ref[...][:, :1] * b_ref[...]
        o = a_ref[0] + alpha * t
        if beta:
            o = o + beta * bp_ref[...]
        o_ref[...] = o

    return pl.pallas_call(
        body,
        grid=(GRID_N,),
        in_specs=[
            pl.BlockSpec((1, BN, w), lambda i: (kslot, i, 0)),
            pl.BlockSpec((BN, w), lambda i: (i, 0)),
            pl.BlockSpec((BN, w), lambda i: (i + GRID_N, 0)),
            pl.BlockSpec((BN, 16), lambda i: (i, 0)),
            pl.BlockSpec((BN, w), lambda i: (i, 0)),
            pl.BlockSpec((BN, w), lambda i: (i, 0)),
        ],
        out_specs=pl.BlockSpec((BN, w), lambda i: (i, 0)),
        out_shape=jax.ShapeDtypeStruct((N_PAD, w), jnp.float32),
    )(a_all, pacc, pacc, diag16, b, bprev)


def _gate_call(a_all, w, h, pacc, diag16, b1, b2, bz, bh, wnext, bnext):
    def body(a_ref, p0_ref, p1_ref, dg_ref, b1_ref, b2_ref, bz_ref, bh_ref,
             wn_ref, bn_ref, o_ref):
        t = p0_ref[...] + p1_ref[...] + dg_ref[...][:, :1] * b1_ref[...]
        s = a_ref[0] + t - b2_ref[...]
        z = jax.nn.sigmoid(s[:, :h] + bz_ref[...])
        ht = jnp.tanh(s[:, h:] + bh_ref[...])
        hid = jnp.maximum((1.0 - z) * ht, 0.0)
        for k in range(wn_ref.shape[0]):
            o_ref[k] = jnp.dot(hid, wn_ref[k],
                               preferred_element_type=jnp.float32) + bn_ref[...]

    kn, ndim = wnext.shape[0], wnext.shape[2]
    return pl.pallas_call(
        body,
        grid=(GRID_N,),
        in_specs=[
            pl.BlockSpec((1, BN, w), lambda i: (0, i, 0)),
            pl.BlockSpec((BN, w), lambda i: (i, 0)),
            pl.BlockSpec((BN, w), lambda i: (i + GRID_N, 0)),
            pl.BlockSpec((BN, 16), lambda i: (i, 0)),
            pl.BlockSpec((BN, w), lambda i: (i, 0)),
            pl.BlockSpec((BN, w), lambda i: (i, 0)),
            pl.BlockSpec((1, h), lambda i: (0, 0)),
            pl.BlockSpec((1, h), lambda i: (0, 0)),
            pl.BlockSpec((kn, h, ndim), lambda i: (0, 0, 0)),
            pl.BlockSpec((1, ndim), lambda i: (0, 0)),
        ],
        out_specs=pl.BlockSpec((kn, BN, ndim), lambda i: (0, i, 0)),
        out_shape=jax.ShapeDtypeStruct((kn, N_PAD, ndim), jnp.float32),
    )(a_all, pacc, pacc, diag16, b1, b2, bz, bh, wnext, bnext)


def _cheb_layer(a_all, w, row2, col2, norm2, diag16):
    prop = _prop64 if w == 64 else _prop32
    b4 = a_all[4]
    p = prop(b4, row2, col2, norm2)
    b3 = _axpy_call(a_all, 3, w, p, diag16, b4, b4, 2.0, 0.0)
    p = prop(b3, row2, col2, norm2)
    b2 = _axpy_call(a_all, 2, w, p, diag16, b3, b4, 2.0, -1.0)
    p = prop(b2, row2, col2, norm2)
    b1 = _axpy_call(a_all, 1, w, p, diag16, b2, b3, 2.0, -1.0)
    p = prop(b1, row2, col2, norm2)
    return p, b1, b2


def kernel(x, edge_index, edge_weight, params):
    f32 = jnp.float32
    row = edge_index[0]
    col = edge_index[1]

    epad = E_PAD - E
    row2 = jnp.pad(row, (0, epad)).reshape(EB, 128)
    col2 = jnp.pad(col, (0, epad)).reshape(EB, 128)
    ew2 = jnp.pad(edge_weight, (0, epad)).reshape(EB, 128)
    x_pad = jnp.pad(x, ((0, N_PAD - N), (0, 0)))

    wcat1 = jnp.stack(
        [jnp.concatenate([params["xz1"]["W"][k], params["xh1"]["W"][k]], axis=1)
         for k in range(K)], axis=0)
    wcat2 = jnp.stack(
        [jnp.concatenate([params["xz2"]["W"][k], params["xh2"]["W"][k]], axis=1)
         for k in range(K)], axis=0)
    bz1 = (params["xz1"]["b"] + params["hz1"]["b"]).reshape(1, H1)
    bh1 = (params["xh1"]["b"] + params["hh1"]["b"]).reshape(1, H1)
    bz2 = (params["xz2"]["b"] + params["hz2"]["b"]).reshape(1, H2D)
    bh2 = (params["xh2"]["b"] + params["hh2"]["b"]).reshape(1, H2D)
    lin_w = params["lin_W"]
    lin_b = params["lin_b"].reshape(1, PERIODS)

    dpart = _deg_call(row2, col2, ew2)
    a1, dis16, diag16 = _prep_call(x_pad, wcat1, dpart)
    norm2 = _norm_call(row2, col2, ew2, dis16)

    p, b1, b2 = _cheb_layer(a1, 64, row2, col2, norm2, diag16)
    a2 = _gate_call(a1, 64, H1, p, diag16, b1, b2, bz1, bh1, wcat2,
                    jnp.zeros((1, wcat2.shape[2]), f32))

    p, b1, b2 = _cheb_layer(a2, 32, row2, col2, norm2, diag16)
    out_pad = _gate_call(a2, 32, H2D, p, diag16, b1, b2, bz2, bh2,
                         lin_w.reshape(1, H2D, PERIODS), lin_b)

    return out_pad[0, :N]


# speedup vs baseline: 14.1537x; 2.5846x over previous
import dataclasses
import functools

import jax
import jax.numpy as jnp
from jax import lax
from jax.experimental import pallas as pl
from jax.experimental.pallas import tpu as pltpu
from jax.experimental.pallas import tpu_sc as plsc

N = 10000
E = 320000
K = 5
F_IN = 128
H1 = 32
H2D = 16
PERIODS = 12

N_PAD = 10240
NW = 32
EPT = 10240
E_PAD = NW * EPT
BPT = EPT // 128
CHUNK = 512
NCHUNK = EPT // CHUNK
BN = 1024
GRID_N = N_PAD // BN

_mesh = plsc.VectorSubcoreMesh(core_axis_name="c", subcore_axis_name="s")

_sc_params = pltpu.CompilerParams(
    needs_layout_passes=False, use_tc_tiling_on_sc=False)


def _iota16():
    return lax.iota(jnp.int32, 16)


def _fullc(c):
    return jnp.full((16,), c, jnp.int32)


EB = E_PAD // 128


def _deg_call(row2, col2, ew2):
    @functools.partial(
        pl.kernel,
        out_type=jax.ShapeDtypeStruct((2 * N_PAD, 16), jnp.float32),
        mesh=_mesh,
        compiler_params=_sc_params,
        scratch_types=[
            pltpu.VMEM_SHARED((N_PAD, 16), jnp.float32),
            pltpu.VMEM((BPT, 128), jnp.int32),
            pltpu.VMEM((BPT, 128), jnp.int32),
            pltpu.VMEM((BPT, 128), jnp.float32),
            pltpu.VMEM((128, 16), jnp.float32),
        ],
    )
    def deg_kernel(row_hbm, col_hbm, ew_hbm, dpart_hbm,
                   acc, rowv, colv, ewv, sbuf):
        cid = lax.axis_index("c")
        sid = lax.axis_index("s")
        wid = cid * 16 + sid
        it = _iota16()
        zero = jnp.zeros((16,), jnp.float32)

        @pl.loop(0, 8)
        def _(g):
            rows = g * 16 + it
            for c in range(16):
                plsc.store_scatter(sbuf, [rows, _fullc(c)], zero)

        for k in range(5):
            pltpu.sync_copy(sbuf, acc.at[pl.ds(sid * 640 + k * 128, 128)])
        plsc.subcore_barrier()

        jbase = wid * BPT
        pltpu.sync_copy(row_hbm.at[pl.ds(jbase, BPT)], rowv)
        pltpu.sync_copy(col_hbm.at[pl.ds(jbase, BPT)], colv)
        pltpu.sync_copy(ew_hbm.at[pl.ds(jbase, BPT)], ewv)

        @pl.loop(0, BPT)
        def _(j):
            @pl.loop(0, 8)
            def _(g):
                s = pl.ds(g * 16, 16)
                rv = rowv.at[j][s]
                cv = colv.at[j][s]
                wv = ewv.at[j][s]
                wz = jnp.where(rv == cv, 0.0, wv)
                rows = g * 16 + it
                for c in range(16):
                    plsc.store_scatter(sbuf, [rows, _fullc(c)], wz)
            pltpu.sync_copy(sbuf, acc.at[rowv.at[j]], add=True)

        plsc.subcore_barrier()
        pltpu.sync_copy(acc.at[pl.ds(sid * 640, 640)],
                        dpart_hbm.at[pl.ds(cid * N_PAD + sid * 640, 640)])

    return deg_kernel(row2, col2, ew2)


def _norm_call(row2, col2, ew2, dis16):
    @functools.partial(
        pl.kernel,
        out_type=jax.ShapeDtypeStruct((EB, 128), jnp.float32),
        mesh=_mesh,
        compiler_params=_sc_params,
        scratch_types=[
            pltpu.VMEM((BPT, 128), jnp.int32),
            pltpu.VMEM((BPT, 128), jnp.int32),
            pltpu.VMEM((BPT, 128), jnp.float32),
            pltpu.VMEM((BPT, 128), jnp.float32),
            pltpu.VMEM((128, 16), jnp.float32),
            pltpu.VMEM((128, 16), jnp.float32),
        ],
    )
    def norm_kernel(row_hbm, col_hbm, ew_hbm, dis_hbm, norm_hbm,
                    rowv, colv, ewv, nout, drb, dcb):
        cid = lax.axis_index("c")
        sid = lax.axis_index("s")
        wid = cid * 16 + sid
        it = _iota16()
        z16 = jnp.zeros((16,), jnp.int32)

        jbase = wid * BPT
        pltpu.sync_copy(row_hbm.at[pl.ds(jbase, BPT)], rowv)
        pltpu.sync_copy(col_hbm.at[pl.ds(jbase, BPT)], colv)
        pltpu.sync_copy(ew_hbm.at[pl.ds(jbase, BPT)], ewv)

        @pl.loop(0, BPT)
        def _(j):
            pltpu.sync_copy(dis_hbm.at[rowv.at[j]], drb)
            pltpu.sync_copy(dis_hbm.at[colv.at[j]], dcb)

            @pl.loop(0, 8)
            def _(g):
                s = pl.ds(g * 16, 16)
                rv = rowv.at[j][s]
                cv = colv.at[j][s]
                wv = ewv.at[j][s]
                wz = jnp.where(rv == cv, 0.0, wv)
                rows = g * 16 + it
                dr = plsc.load_gather(drb, [rows, z16])
                dc = plsc.load_gather(dcb, [rows, z16])
                nout.at[j][s] = -(dr * wz * dc)

        pltpu.sync_copy(nout, norm_hbm.at[pl.ds(jbase, BPT)])

    return norm_kernel(row2, col2, ew2, dis16)


def _make_prop(w):
    nbuf = 2 if w == 64 else 4

    @functools.partial(
        pl.kernel,
        out_type=jax.ShapeDtypeStruct((2 * N_PAD, w), jnp.float32),
        mesh=_mesh,
        compiler_params=_sc_params,
        scratch_types=(
            [pltpu.VMEM_SHARED((N_PAD, w), jnp.float32),
             pltpu.VMEM((BPT, 128), jnp.int32),
             pltpu.VMEM((BPT, 128), jnp.int32),
             pltpu.VMEM((BPT, 128), jnp.float32)]
            + [pltpu.VMEM((128, w), jnp.float32)] * (2 * nbuf)
            + [pltpu.SemaphoreType.DMA] * (2 * nbuf)
        ),
    )
    def prop_kernel(b_hbm, row_hbm, col_hbm, norm_hbm, pacc_hbm,
                    acc, rowv, colv, normv, *bufsem):
        gbuf = bufsem[:nbuf]
        sbuf = bufsem[nbuf:2 * nbuf]
        gsem = bufsem[2 * nbuf:3 * nbuf]
        ssem = bufsem[3 * nbuf:]
        cid = lax.axis_index("c")
        sid = lax.axis_index("s")
        wid = cid * 16 + sid
        it = _iota16()
        zero = jnp.zeros((16,), jnp.float32)

        @pl.loop(0, 8)
        def _(g):
            rows = g * 16 + it
            for c in range(w):
                plsc.store_scatter(gbuf[0], [rows, _fullc(c)], zero)

        for k in range(5):
            pltpu.sync_copy(gbuf[0], acc.at[pl.ds(sid * 640 + k * 128, 128)])
        plsc.subcore_barrier()

        jbase = wid * BPT
        pltpu.sync_copy(row_hbm.at[pl.ds(jbase, BPT)], rowv)
        pltpu.sync_copy(col_hbm.at[pl.ds(jbase, BPT)], colv)
        pltpu.sync_copy(norm_hbm.at[pl.ds(jbase, BPT)], normv)

        zi = jnp.zeros((16,), jnp.int32)

        def scale(j, src, dst):
            nb = normv.at[j]

            @pl.loop(0, 128, unroll=8)
            def _(e):
                ns = plsc.load_gather(nb, [e + zi])
                for cc in range(w // 16):
                    s = pl.ds(cc * 16, 16)
                    dst.at[e][s] = src.at[e][s] * ns

        for b in range(nbuf):
            pltpu.async_copy(b_hbm.at[rowv.at[b]], gbuf[b], gsem[b])

        nit = BPT // nbuf

        @pl.loop(0, nit)
        def _(jj):
            for b in range(nbuf):
                j = jj * nbuf + b
                pltpu.make_async_copy(
                    b_hbm.at[rowv.at[j]], gbuf[b], gsem[b]).wait()

                @pl.when(jj > 0)
                def _():
                    pltpu.make_async_copy(
                        sbuf[b], acc.at[colv.at[j - nbuf]], ssem[b]).wait()

                scale(j, gbuf[b], sbuf[b])

                @pl.when(jj < nit - 1)
                def _():
                    pltpu.async_copy(
                        b_hbm.at[rowv.at[j + nbuf]], gbuf[b], gsem[b])

                pltpu.async_copy(sbuf[b], acc.at[colv.at[j]], ssem[b],
                                 add=True)

        for b in range(nbuf):
            pltpu.make_async_copy(
                sbuf[b], acc.at[colv.at[BPT - nbuf + b]], ssem[b]).wait()

        plsc.subcore_barrier()
        pltpu.sync_copy(acc.at[pl.ds(sid * 640, 640)],
                        pacc_hbm.at[pl.ds(cid * N_PAD + sid * 640, 640)])

    return prop_kernel


_prop64 = _make_prop(64)
_prop32 = _make_prop(32)


def _prep_call(x_pad, wcat, p0p1):
    def body(x_ref, w_ref, p0_ref, p1_ref, a_ref, dis_ref, diag_ref):
        for k in range(K):
            a_ref[k] = jnp.dot(x_ref[...], w_ref[k],
                               preferred_element_type=jnp.float32)
        d = p0_ref[...] + p1_ref[...]
        pos = d > 0.0
        dis_ref[...] = jnp.where(pos, lax.rsqrt(jnp.where(pos, d, 1.0)), 0.0)
        diag_ref[...] = jnp.where(pos, 0.0, -1.0)

    fin, wdim = wcat.shape[1], wcat.shape[2]
    return pl.pallas_call(
        body,
        grid=(GRID_N,),
        in_specs=[
            pl.BlockSpec((BN, fin), lambda i: (i, 0)),
            pl.BlockSpec((K, fin, wdim), lambda i: (0, 0, 0)),
            pl.BlockSpec((BN, 16), lambda i: (i, 0)),
            pl.BlockSpec((BN, 16), lambda i: (i + GRID_N, 0)),
        ],
        out_specs=[
            pl.BlockSpec((K, BN, wdim), lambda i: (0, i, 0)),
            pl.BlockSpec((BN, 16), lambda i: (i, 0)),
            pl.BlockSpec((BN, 16), lambda i: (i, 0)),
        ],
        out_shape=[
            jax.ShapeDtypeStruct((K, N_PAD, wdim), jnp.float32),
            jax.ShapeDtypeStruct((N_PAD, 16), jnp.float32),
            jax.ShapeDtypeStruct((N_PAD, 16), jnp.float32),
        ],
    )(x_pad, wcat, p0p1, p0p1)


def _axpy_call(a_all, kslot, w, pacc, diag16, b, bprev, alpha, beta):
    def body(a_ref, p0_ref, p1_ref, dg_ref, b_ref, bp_ref, o_ref):
        t = p0_ref[...] + p1_ref[...] + dg_ref[...][:, :1] * b_ref[...]
        o = a_ref[0] + alpha * t
        if beta:
            o = o + beta * bp_ref[...]
        o_ref[...] = o

    return pl.pallas_call(
        body,
        grid=(GRID_N,),
        in_specs=[
            pl.BlockSpec((1, BN, w), lambda i: (kslot, i, 0)),
            pl.BlockSpec((BN, w), lambda i: (i, 0)),
            pl.BlockSpec((BN, w), lambda i: (i + GRID_N, 0)),
            pl.BlockSpec((BN, 16), lambda i: (i, 0)),
            pl.BlockSpec((BN, w), lambda i: (i, 0)),
            pl.BlockSpec((BN, w), lambda i: (i, 0)),
        ],
        out_specs=pl.BlockSpec((BN, w), lambda i: (i, 0)),
        out_shape=jax.ShapeDtypeStruct((N_PAD, w), jnp.float32),
    )(a_all, pacc, pacc, diag16, b, bprev)


def _gate_call(a_all, w, h, pacc, diag16, b1, b2, bz, bh, wnext, bnext):
    def body(a_ref, p0_ref, p1_ref, dg_ref, b1_ref, b2_ref, bz_ref, bh_ref,
             wn_ref, bn_ref, o_ref):
        t = p0_ref[...] + p1_ref[...] + dg_ref[...][:, :1] * b1_ref[...]
        s = a_ref[0] + t - b2_ref[...]
        z = jax.nn.sigmoid(s[:, :h] + bz_ref[...])
        ht = jnp.tanh(s[:, h:] + bh_ref[...])
        hid = jnp.maximum((1.0 - z) * ht, 0.0)
        for k in range(wn_ref.shape[0]):
            o_ref[k] = jnp.dot(hid, wn_ref[k],
                               preferred_element_type=jnp.float32) + bn_ref[...]

    kn, ndim = wnext.shape[0], wnext.shape[2]
    return pl.pallas_call(
        body,
        grid=(GRID_N,),
        in_specs=[
            pl.BlockSpec((1, BN, w), lambda i: (0, i, 0)),
            pl.BlockSpec((BN, w), lambda i: (i, 0)),
            pl.BlockSpec((BN, w), lambda i: (i + GRID_N, 0)),
            pl.BlockSpec((BN, 16), lambda i: (i, 0)),
            pl.BlockSpec((BN, w), lambda i: (i, 0)),
            pl.BlockSpec((BN, w), lambda i: (i, 0)),
            pl.BlockSpec((1, h), lambda i: (0, 0)),
            pl.BlockSpec((1, h), lambda i: (0, 0)),
            pl.BlockSpec((kn, h, ndim), lambda i: (0, 0, 0)),
            pl.BlockSpec((1, ndim), lambda i: (0, 0)),
        ],
        out_specs=pl.BlockSpec((kn, BN, ndim), lambda i: (0, i, 0)),
        out_shape=jax.ShapeDtypeStruct((kn, N_PAD, ndim), jnp.float32),
    )(a_all, pacc, pacc, diag16, b1, b2, bz, bh, wnext, bnext)


def _cheb_layer(a_all, w, row2, col2, norm2, diag16):
    prop = _prop64 if w == 64 else _prop32
    b4 = a_all[4]
    p = prop(b4, row2, col2, norm2)
    b3 = _axpy_call(a_all, 3, w, p, diag16, b4, b4, 2.0, 0.0)
    p = prop(b3, row2, col2, norm2)
    b2 = _axpy_call(a_all, 2, w, p, diag16, b3, b4, 2.0, -1.0)
    p = prop(b2, row2, col2, norm2)
    b1 = _axpy_call(a_all, 1, w, p, diag16, b2, b3, 2.0, -1.0)
    p = prop(b1, row2, col2, norm2)
    return p, b1, b2


def kernel(x, edge_index, edge_weight, params):
    f32 = jnp.float32
    row = edge_index[0]
    col = edge_index[1]

    epad = E_PAD - E
    row2 = jnp.pad(row, (0, epad)).reshape(EB, 128)
    col2 = jnp.pad(col, (0, epad)).reshape(EB, 128)
    ew2 = jnp.pad(edge_weight, (0, epad)).reshape(EB, 128)
    x_pad = jnp.pad(x, ((0, N_PAD - N), (0, 0)))

    wcat1 = jnp.stack(
        [jnp.concatenate([params["xz1"]["W"][k], params["xh1"]["W"][k]], axis=1)
         for k in range(K)], axis=0)
    wcat2 = jnp.stack(
        [jnp.concatenate([params["xz2"]["W"][k], params["xh2"]["W"][k]], axis=1)
         for k in range(K)], axis=0)
    bz1 = (params["xz1"]["b"] + params["hz1"]["b"]).reshape(1, H1)
    bh1 = (params["xh1"]["b"] + params["hh1"]["b"]).reshape(1, H1)
    bz2 = (params["xz2"]["b"] + params["hz2"]["b"]).reshape(1, H2D)
    bh2 = (params["xh2"]["b"] + params["hh2"]["b"]).reshape(1, H2D)
    lin_w = params["lin_W"]
    lin_b = params["lin_b"].reshape(1, PERIODS)

    dpart = _deg_call(row2, col2, ew2)
    a1, dis16, diag16 = _prep_call(x_pad, wcat1, dpart)
    norm2 = _norm_call(row2, col2, ew2, dis16)

    p, b1, b2 = _cheb_layer(a1, 64, row2, col2, norm2, diag16)
    a2 = _gate_call(a1, 64, H1, p, diag16, b1, b2, bz1, bh1, wcat2,
                    jnp.zeros((1, wcat2.shape[2]), f32))

    p, b1, b2 = _cheb_layer(a2, 32, row2, col2, norm2, diag16)
    out_pad = _gate_call(a2, 32, H2D, p, diag16, b1, b2, bz2, bh2,
                         lin_w.reshape(1, H2D, PERIODS), lin_b)

    return out_pad[0, :N]
